# baseline (speedup 1.0000x reference)
import functools

import jax
import jax.numpy as jnp
from jax import lax
from jax.experimental import pallas as pl
from jax.experimental.pallas import tpu as pltpu
from jax.experimental.pallas import tpu_sc as plsc

_ROWS = 98304
_LANES = 128
_NW = 32
_ROWS_PER_W = _ROWS // _NW
_CHUNK = 256
_NCHUNK = _ROWS_PER_W // _CHUNK


def _make_sc_copy():
    mesh = plsc.VectorSubcoreMesh(core_axis_name="c", subcore_axis_name="s")

    @functools.partial(
        pl.kernel,
        mesh=mesh,
        out_type=jax.ShapeDtypeStruct((_ROWS, _LANES), jnp.float32),
        scratch_types=[
            pltpu.VMEM((_CHUNK, _LANES), jnp.float32),
            pltpu.VMEM((_CHUNK, _LANES), jnp.float32),
            pltpu.SemaphoreType.DMA,
            pltpu.SemaphoreType.DMA,
        ],
    )
    def sc_copy(x_hbm, o_hbm, buf0, buf1, sem0, sem1):
        wid = lax.axis_index("s") * 2 + lax.axis_index("c")
        base = wid * _ROWS_PER_W
        bufs = (buf0, buf1)
        sems = (sem0, sem1)
        pltpu.make_async_copy(
            x_hbm.at[pl.ds(base, _CHUNK)], bufs[0], sems[0]).start()
        for g in range(_NCHUNK):
            cur = g % 2
            nxt = (g + 1) % 2
            if g + 1 < _NCHUNK:
                pltpu.make_async_copy(
                    x_hbm.at[pl.ds(base + (g + 1) * _CHUNK, _CHUNK)],
                    bufs[nxt], sems[nxt]).start()
            pltpu.make_async_copy(
                x_hbm.at[pl.ds(base + g * _CHUNK, _CHUNK)],
                bufs[cur], sems[cur]).wait()
            pltpu.sync_copy(bufs[cur], o_hbm.at[pl.ds(base + g * _CHUNK, _CHUNK)])

    return sc_copy


_sc_copy = _make_sc_copy()


def kernel(items):
    n = items.shape[0]
    chunks = n // 128
    flat = (jnp.transpose(items, (1, 3, 0, 2))
            .reshape(2, 3, chunks, 128, 2)
            .transpose(0, 1, 2, 4, 3)
            .reshape(_ROWS, _LANES))
    out = _sc_copy(flat)
    return (out.reshape(2, 3, chunks, 2, 128)
            .transpose(0, 1, 2, 4, 3)
            .reshape(2, 3, n, 2)
            .transpose(2, 0, 3, 1))


# speedup vs baseline: 1.6605x; 1.6605x over previous
import jax
import jax.numpy as jnp
from jax.experimental import pallas as pl


def _copy_body(x_ref, o_ref):
    o_ref[...] = x_ref[...]


def kernel(items):
    n = items.shape[0]
    chunks = n // 128
    rows = 2 * 3 * chunks * 2
    flat = (jnp.transpose(items, (1, 3, 0, 2))
            .reshape(2, 3, chunks, 128, 2)
            .transpose(0, 1, 2, 4, 3)
            .reshape(rows, 128))
    block_rows = 8192
    out = pl.pallas_call(
        _copy_body,
        grid=(rows // block_rows,),
        in_specs=[pl.BlockSpec((block_rows, 128), lambda i: (i, 0))],
        out_specs=pl.BlockSpec((block_rows, 128), lambda i: (i, 0)),
        out_shape=jax.ShapeDtypeStruct((rows, 128), jnp.float32),
    )(flat)
    return (out.reshape(2, 3, chunks, 2, 128)
            .transpose(0, 1, 2, 4, 3)
            .reshape(2, 3, n, 2)
            .transpose(2, 0, 3, 1))


# speedup vs baseline: 1.7370x; 1.0460x over previous
import jax
import jax.numpy as jnp
from jax.experimental import pallas as pl


def _copy_body(x_ref, o_ref):
    o_ref[...] = x_ref[...]


def kernel(items):
    n = items.shape[0]
    chunks = n // 128
    rows = 2 * 3 * chunks * 2
    flat = (jnp.transpose(items, (1, 3, 0, 2))
            .reshape(2, 3, chunks, 128, 2)
            .transpose(0, 1, 2, 4, 3)
            .reshape(rows, 128))
    block_rows = 16384
    out = pl.pallas_call(
        _copy_body,
        grid=(rows // block_rows,),
        in_specs=[pl.BlockSpec((block_rows, 128), lambda i: (i, 0))],
        out_specs=pl.BlockSpec((block_rows, 128), lambda i: (i, 0)),
        out_shape=jax.ShapeDtypeStruct((rows, 128), jnp.float32),
    )(flat)
    return (out.reshape(2, 3, chunks, 2, 128)
            .transpose(0, 1, 2, 4, 3)
            .reshape(2, 3, n, 2)
            .transpose(2, 0, 3, 1))
